# Initial kernel scaffold; baseline (speedup 1.0000x reference)
import jax
import jax.numpy as jnp
from jax.experimental import pallas as pl


def kernel(nodes, neigh_pos, neigh_neg, feat_table, W_bal, a_bal, W_unbal, a_unbal):
    raise NotImplementedError("write your pallas kernel here")


# speedup vs baseline: 1.1779x; 1.1779x over previous
import jax
import jax.numpy as jnp
from jax import lax
from jax.experimental import pallas as pl
from jax.experimental.pallas import tpu as pltpu
from jax.experimental.pallas import tpu_sc as plsc

_N = 100000
_DIN = 128
_DOUT = 32
_B = 10000
_DEG = 16
_ALPHA = 0.2

_NC, _NS = 2, 16
_NW = _NC * _NS
_LANES = 128

_SELF_IR = 3
_SELF_PW = _SELF_IR * _LANES
_SELF_PAD = _NW * _SELF_PW
_NB_IR = 40
_NB_PW = _NB_IR * _LANES
_NB_PAD = _NW * _NB_PW
_NB_CHUNKS = 5
_CH_IR = 8
_CHUNK = _CH_IR * _LANES


def _leaky(x):
    return jnp.where(x >= 0, x, _ALPHA * x)


_PROJ_ROWS = 2000


def _proj_body(f_ref, wb_ref, wu_ref, hb_ref, hu_ref):
    f = f_ref[...]
    hb_ref[...] = jnp.dot(f, wb_ref[...], preferred_element_type=jnp.float32)
    hu_ref[...] = jnp.dot(f, wu_ref[...], preferred_element_type=jnp.float32)


def _project(feat, wb, wu):
    return pl.pallas_call(
        _proj_body,
        grid=(_N // _PROJ_ROWS,),
        in_specs=[
            pl.BlockSpec((_PROJ_ROWS, _DIN), lambda i: (i, 0)),
            pl.BlockSpec((_DIN, _DOUT), lambda i: (0, 0)),
            pl.BlockSpec((_DIN, _DOUT), lambda i: (0, 0)),
        ],
        out_specs=[
            pl.BlockSpec((_PROJ_ROWS, _DOUT), lambda i: (i, 0)),
            pl.BlockSpec((_PROJ_ROWS, _DOUT), lambda i: (i, 0)),
        ],
        out_shape=[
            jax.ShapeDtypeStruct((_N, _DOUT), jnp.float32),
            jax.ShapeDtypeStruct((_N, _DOUT), jnp.float32),
        ],
    )(feat, wb, wu)


def _gather_body(hb, hu, iself, ipos, ineg,
                 sb, su, nbp, nbn,
                 idx_s, idx_n, rows, srows, sem):
    wid = lax.axis_index("s") * _NC + lax.axis_index("c")
    s_base = pl.multiple_of(wid * _SELF_PW, _SELF_PW)
    n_base = pl.multiple_of(wid * _NB_PW, _NB_PW)

    pltpu.sync_copy(iself.at[pl.ds(s_base, _SELF_PW)], idx_s)
    for tbl, out in ((hb, sb), (hu, su)):
        for j in range(_SELF_IR):
            pltpu.async_copy(tbl.at[idx_s.at[pl.ds(j * _LANES, _LANES)]],
                             srows.at[pl.ds(j * _LANES, _LANES)], sem).wait()
        pltpu.sync_copy(srows, out.at[pl.ds(s_base, _SELF_PW)])

    for itab, tbl, out in ((ipos, hb, nbp), (ineg, hu, nbn)):
        pltpu.sync_copy(itab.at[pl.ds(n_base, _NB_PW)], idx_n)

        def chunk(c, carry):
            c_off = pl.multiple_of(c * _CHUNK, _CHUNK)
            for j in range(_CH_IR):
                pltpu.async_copy(
                    tbl.at[idx_n.at[pl.ds(c_off + j * _LANES, _LANES)]],
                    rows.at[pl.ds(j * _LANES, _LANES)], sem).wait()
            pltpu.sync_copy(
                rows, out.at[pl.ds(n_base + c_off, _CHUNK)])
            return carry

        lax.fori_loop(0, _NB_CHUNKS, chunk, 0)


def _gather(hb, hu, iself, ipos, ineg):
    mesh = plsc.VectorSubcoreMesh(core_axis_name="c", subcore_axis_name="s")
    f = pl.kernel(
        _gather_body,
        mesh=mesh,
        out_type=[
            jax.ShapeDtypeStruct((_SELF_PAD, _DOUT), jnp.float32),
            jax.ShapeDtypeStruct((_SELF_PAD, _DOUT), jnp.float32),
            jax.ShapeDtypeStruct((_NB_PAD, _DOUT), jnp.float32),
            jax.ShapeDtypeStruct((_NB_PAD, _DOUT), jnp.float32),
        ],
        scratch_types=[
            pltpu.VMEM((_SELF_PW,), jnp.int32),
            pltpu.VMEM((_NB_PW,), jnp.int32),
            pltpu.VMEM((_CHUNK, _DOUT), jnp.float32),
            pltpu.VMEM((_SELF_PW, _DOUT), jnp.float32),
            pltpu.SemaphoreType.DMA,
        ],
        compiler_params=pltpu.CompilerParams(use_tc_tiling_on_sc=False),
    )
    return f(hb, hu, iself, ipos, ineg)


_BR = 400
_NBLK = _B // _BR


def _attn_body(hsb, hsu, npos, nneg, a1b, a2b, a1u, a2u,
               xb, xu, rtb, rtu, dnb, dnu):
    j = pl.program_id(1)

    @pl.when(j == 0)
    def _init():
        for hs, a1, a2, rt, dn, x in ((hsb, a1b, a2b, rtb, dnb, xb),
                                      (hsu, a1u, a2u, rtu, dnu, xu)):
            h = hs[...]
            r = jnp.sum(h * a1[...], axis=1, keepdims=True)
            e = jnp.exp(-_leaky(r + jnp.sum(h * a2[...], axis=1,
                                            keepdims=True)))
            rt[...] = r
            dn[...] = e
            x[...] = e * h

    for nb, a2, rt, dn, x in ((npos, a2b, rtb, dnb, xb),
                              (nneg, a2u, rtu, dnu, xu)):
        h = nb[...]
        e = jnp.exp(-_leaky(rt[...] + jnp.sum(h * a2[...], axis=1,
                                              keepdims=True)))
        dn[...] += e
        x[...] += e * h

    @pl.when(j == _DEG - 1)
    def _final():
        for x, dn in ((xb, dnb), (xu, dnu)):
            v = x[...] / (dn[...] + 1e-16)
            x[...] = jnp.where(v > 0, v, jnp.exp(v) - 1.0)


def _attention(hsb, hsu, npos, nneg, a1b, a2b, a1u, a2u):
    return pl.pallas_call(
        _attn_body,
        grid=(_NBLK, _DEG),
        in_specs=[
            pl.BlockSpec((_BR, _DOUT), lambda b, j: (b, 0)),
            pl.BlockSpec((_BR, _DOUT), lambda b, j: (b, 0)),
            pl.BlockSpec((_BR, _DOUT), lambda b, j: (j * _NBLK + b, 0)),
            pl.BlockSpec((_BR, _DOUT), lambda b, j: (j * _NBLK + b, 0)),
            pl.BlockSpec((1, _DOUT), lambda b, j: (0, 0)),
            pl.BlockSpec((1, _DOUT), lambda b, j: (0, 0)),
            pl.BlockSpec((1, _DOUT), lambda b, j: (0, 0)),
            pl.BlockSpec((1, _DOUT), lambda b, j: (0, 0)),
        ],
        out_specs=[
            pl.BlockSpec((_BR, _DOUT), lambda b, j: (b, 0)),
            pl.BlockSpec((_BR, _DOUT), lambda b, j: (b, 0)),
        ],
        out_shape=[
            jax.ShapeDtypeStruct((_B, _DOUT), jnp.float32),
            jax.ShapeDtypeStruct((_B, _DOUT), jnp.float32),
        ],
        scratch_shapes=[pltpu.VMEM((_BR, 1), jnp.float32)] * 4,
    )(hsb, hsu, npos, nneg, a1b, a2b, a1u, a2u)


def kernel(nodes, neigh_pos, neigh_neg, feat_table,
           W_bal, a_bal, W_unbal, a_unbal):
    hb, hu = _project(feat_table, W_bal, W_unbal)

    zs = jnp.zeros(_SELF_PAD - _B, jnp.int32)
    zn = jnp.zeros(_NB_PAD - _B * _DEG, jnp.int32)
    iself = jnp.concatenate([nodes, zs])
    ipos = jnp.concatenate([neigh_pos.T.reshape(-1), zn])
    ineg = jnp.concatenate([neigh_neg.T.reshape(-1), zn])

    sb, su, nbp, nbn = _gather(hb, hu, iself, ipos, ineg)

    a1b, a2b = a_bal[:, :_DOUT], a_bal[:, _DOUT:]
    a1u, a2u = a_unbal[:, :_DOUT], a_unbal[:, _DOUT:]
    return _attention(sb[:_B], su[:_B], nbp, nbn, a1b, a2b, a1u, a2u)


# speedup vs baseline: 1.6103x; 1.3671x over previous
import jax
import jax.numpy as jnp
from jax import lax
from jax.experimental import pallas as pl
from jax.experimental.pallas import tpu as pltpu
from jax.experimental.pallas import tpu_sc as plsc

_N = 100000
_DIN = 128
_DOUT = 32
_B = 10000
_DEG = 16
_ALPHA = 0.2

_NC, _NS = 2, 16
_NW = _NC * _NS
_LANES = 128

_SELF_IR = 3
_SELF_PW = _SELF_IR * _LANES
_SELF_PAD = _NW * _SELF_PW
_NB_IR = 40
_NB_PW = _NB_IR * _LANES
_NB_PAD = _NW * _NB_PW
_CH_IR = 10
_CHUNK = _CH_IR * _LANES


def _leaky(x):
    return jnp.where(x >= 0, x, _ALPHA * x)


_PROJ_ROWS = 2000


def _proj_body(f_ref, wb_ref, wu_ref, hb_ref, hu_ref):
    f = f_ref[...]
    hb_ref[...] = jnp.dot(f, wb_ref[...], preferred_element_type=jnp.float32)
    hu_ref[...] = jnp.dot(f, wu_ref[...], preferred_element_type=jnp.float32)


def _project(feat, wb, wu):
    return pl.pallas_call(
        _proj_body,
        grid=(_N // _PROJ_ROWS,),
        in_specs=[
            pl.BlockSpec((_PROJ_ROWS, _DIN), lambda i: (i, 0)),
            pl.BlockSpec((_DIN, _DOUT), lambda i: (0, 0)),
            pl.BlockSpec((_DIN, _DOUT), lambda i: (0, 0)),
        ],
        out_specs=[
            pl.BlockSpec((_PROJ_ROWS, _DOUT), lambda i: (i, 0)),
            pl.BlockSpec((_PROJ_ROWS, _DOUT), lambda i: (i, 0)),
        ],
        out_shape=[
            jax.ShapeDtypeStruct((_N, _DOUT), jnp.float32),
            jax.ShapeDtypeStruct((_N, _DOUT), jnp.float32),
        ],
    )(feat, wb, wu)


def _fire(tbl, idx, i_off, rows, sem):
    return [
        pltpu.async_copy(tbl.at[idx.at[pl.ds(i_off + j * _LANES, _LANES)]],
                         rows.at[pl.ds(j * _LANES, _LANES)], sem)
        for j in range(_CH_IR)
    ]


def _gather_body(hb, hu, iself, ipos, ineg,
                 sb, su, nbp, nbn,
                 idx_s, idx_n, rows_a, rows_b, sem_a, sem_b):
    wid = lax.axis_index("s") * _NC + lax.axis_index("c")
    s_base = pl.multiple_of(wid * _SELF_PW, _SELF_PW)
    n_base = pl.multiple_of(wid * _NB_PW, _NB_PW)

    pltpu.sync_copy(iself.at[pl.ds(s_base, _SELF_PW)], idx_s)
    da = [pltpu.async_copy(hb.at[idx_s.at[pl.ds(j * _LANES, _LANES)]],
                           rows_a.at[pl.ds(j * _LANES, _LANES)], sem_a)
          for j in range(_SELF_IR)]
    db = [pltpu.async_copy(hu.at[idx_s.at[pl.ds(j * _LANES, _LANES)]],
                           rows_b.at[pl.ds(j * _LANES, _LANES)], sem_b)
          for j in range(_SELF_IR)]
    for d in da:
        d.wait()
    pltpu.sync_copy(rows_a.at[pl.ds(0, _SELF_PW)],
                    sb.at[pl.ds(s_base, _SELF_PW)])
    for d in db:
        d.wait()
    pltpu.sync_copy(rows_b.at[pl.ds(0, _SELF_PW)],
                    su.at[pl.ds(s_base, _SELF_PW)])

    for itab, tbl, out in ((ipos, hb, nbp), (ineg, hu, nbn)):
        pltpu.sync_copy(itab.at[pl.ds(n_base, _NB_PW)], idx_n)

        def pair(p, carry):
            off_a = pl.multiple_of(p * (2 * _CHUNK), 2 * _CHUNK)
            off_b = off_a + _CHUNK
            da = _fire(tbl, idx_n, off_a, rows_a, sem_a)
            db = _fire(tbl, idx_n, off_b, rows_b, sem_b)
            for d in da:
                d.wait()
            pltpu.sync_copy(rows_a, out.at[pl.ds(n_base + off_a, _CHUNK)])
            for d in db:
                d.wait()
            pltpu.sync_copy(rows_b, out.at[pl.ds(n_base + off_b, _CHUNK)])
            return carry

        lax.fori_loop(0, _NB_PW // (2 * _CHUNK), pair, 0)


def _gather(hb, hu, iself, ipos, ineg):
    mesh = plsc.VectorSubcoreMesh(core_axis_name="c", subcore_axis_name="s")
    f = pl.kernel(
        _gather_body,
        mesh=mesh,
        out_type=[
            jax.ShapeDtypeStruct((_SELF_PAD, _DOUT), jnp.float32),
            jax.ShapeDtypeStruct((_SELF_PAD, _DOUT), jnp.float32),
            jax.ShapeDtypeStruct((_NB_PAD, _DOUT), jnp.float32),
            jax.ShapeDtypeStruct((_NB_PAD, _DOUT), jnp.float32),
        ],
        scratch_types=[
            pltpu.VMEM((_SELF_PW,), jnp.int32),
            pltpu.VMEM((_NB_PW,), jnp.int32),
            pltpu.VMEM((_CHUNK, _DOUT), jnp.float32),
            pltpu.VMEM((_CHUNK, _DOUT), jnp.float32),
            pltpu.SemaphoreType.DMA,
            pltpu.SemaphoreType.DMA,
        ],
        compiler_params=pltpu.CompilerParams(use_tc_tiling_on_sc=False),
    )
    return f(hb, hu, iself, ipos, ineg)


_BR = 2000
_NBLK = _B // _BR


def _attn_body(hsb, hsu, npos, nneg, a1b, a2b, a1u, a2u,
               xb, xu, rtb, rtu, dnb, dnu):
    j = pl.program_id(1)

    @pl.when(j == 0)
    def _init():
        for hs, a1, a2, rt, dn, x in ((hsb, a1b, a2b, rtb, dnb, xb),
                                      (hsu, a1u, a2u, rtu, dnu, xu)):
            h = hs[...]
            r = jnp.sum(h * a1[...], axis=1, keepdims=True)
            e = jnp.exp(-_leaky(r + jnp.sum(h * a2[...], axis=1,
                                            keepdims=True)))
            rt[...] = r
            dn[...] = e
            x[...] = e * h

    for nb, a2, rt, dn, x in ((npos, a2b, rtb, dnb, xb),
                              (nneg, a2u, rtu, dnu, xu)):
        h = nb[...]
        e = jnp.exp(-_leaky(rt[...] + jnp.sum(h * a2[...], axis=1,
                                              keepdims=True)))
        dn[...] += e
        x[...] += e * h

    @pl.when(j == _DEG - 1)
    def _final():
        for x, dn in ((xb, dnb), (xu, dnu)):
            v = x[...] / (dn[...] + 1e-16)
            x[...] = jnp.where(v > 0, v, jnp.exp(v) - 1.0)


def _attention(hsb, hsu, npos, nneg, a1b, a2b, a1u, a2u):
    return pl.pallas_call(
        _attn_body,
        grid=(_NBLK, _DEG),
        in_specs=[
            pl.BlockSpec((_BR, _DOUT), lambda b, j: (b, 0)),
            pl.BlockSpec((_BR, _DOUT), lambda b, j: (b, 0)),
            pl.BlockSpec((_BR, _DOUT), lambda b, j: (j * _NBLK + b, 0)),
            pl.BlockSpec((_BR, _DOUT), lambda b, j: (j * _NBLK + b, 0)),
            pl.BlockSpec((1, _DOUT), lambda b, j: (0, 0)),
            pl.BlockSpec((1, _DOUT), lambda b, j: (0, 0)),
            pl.BlockSpec((1, _DOUT), lambda b, j: (0, 0)),
            pl.BlockSpec((1, _DOUT), lambda b, j: (0, 0)),
        ],
        out_specs=[
            pl.BlockSpec((_BR, _DOUT), lambda b, j: (b, 0)),
            pl.BlockSpec((_BR, _DOUT), lambda b, j: (b, 0)),
        ],
        out_shape=[
            jax.ShapeDtypeStruct((_B, _DOUT), jnp.float32),
            jax.ShapeDtypeStruct((_B, _DOUT), jnp.float32),
        ],
        scratch_shapes=[pltpu.VMEM((_BR, 1), jnp.float32)] * 4,
    )(hsb, hsu, npos, nneg, a1b, a2b, a1u, a2u)


def kernel(nodes, neigh_pos, neigh_neg, feat_table,
           W_bal, a_bal, W_unbal, a_unbal):
    hb, hu = _project(feat_table, W_bal, W_unbal)

    zs = jnp.zeros(_SELF_PAD - _B, jnp.int32)
    zn = jnp.zeros(_NB_PAD - _B * _DEG, jnp.int32)
    iself = jnp.concatenate([nodes, zs])
    ipos = jnp.concatenate([neigh_pos.T.reshape(-1), zn])
    ineg = jnp.concatenate([neigh_neg.T.reshape(-1), zn])

    sb, su, nbp, nbn = _gather(hb, hu, iself, ipos, ineg)

    a1b, a2b = a_bal[:, :_DOUT], a_bal[:, _DOUT:]
    a1u, a2u = a_unbal[:, :_DOUT], a_unbal[:, _DOUT:]
    return _attention(sb[:_B], su[:_B], nbp, nbn, a1b, a2b, a1u, a2u)


# speedup vs baseline: 2.4303x; 1.5092x over previous
import jax
import jax.numpy as jnp
from jax import lax
from jax.experimental import pallas as pl
from jax.experimental.pallas import tpu as pltpu
from jax.experimental.pallas import tpu_sc as plsc

_N = 100000
_DIN = 128
_DOUT = 32
_B = 10000
_DEG = 16
_ALPHA = 0.2

_NC, _NS = 2, 16
_NW = _NC * _NS
_LANES = 128

_SELF_IR = 3
_SELF_PW = _SELF_IR * _LANES
_SELF_PAD = _NW * _SELF_PW
_NB_IR = 40
_NB_PW = _NB_IR * _LANES
_NB_PAD = _NW * _NB_PW
_CH_IR = 10
_CHUNK = _CH_IR * _LANES


def _leaky(x):
    return jnp.where(x >= 0, x, _ALPHA * x)


_PROJ_ROWS = 2000


def _proj_body(f_ref, wb_ref, wu_ref, hb_ref, hu_ref):
    f = f_ref[...]
    hb_ref[...] = jnp.dot(f, wb_ref[...], preferred_element_type=jnp.float32)
    hu_ref[...] = jnp.dot(f, wu_ref[...], preferred_element_type=jnp.float32)


def _project(feat, wb, wu):
    return pl.pallas_call(
        _proj_body,
        grid=(_N // _PROJ_ROWS,),
        in_specs=[
            pl.BlockSpec((_PROJ_ROWS, _DIN), lambda i: (i, 0)),
            pl.BlockSpec((_DIN, _DOUT), lambda i: (0, 0)),
            pl.BlockSpec((_DIN, _DOUT), lambda i: (0, 0)),
        ],
        out_specs=[
            pl.BlockSpec((_PROJ_ROWS, _DOUT), lambda i: (i, 0)),
            pl.BlockSpec((_PROJ_ROWS, _DOUT), lambda i: (i, 0)),
        ],
        out_shape=[
            jax.ShapeDtypeStruct((_N, _DOUT), jnp.float32),
            jax.ShapeDtypeStruct((_N, _DOUT), jnp.float32),
        ],
    )(feat, wb, wu)


def _fire(tbl, idx, i_off, rows, sem):
    return [
        pltpu.async_copy(tbl.at[idx.at[pl.ds(i_off + j * _LANES, _LANES)]],
                         rows.at[pl.ds(j * _LANES, _LANES)], sem)
        for j in range(_CH_IR)
    ]


def _gather_body(hb, hu, iself, ipos, ineg,
                 sb, su, nbp, nbn,
                 idx_s, idx_n, rows_a, rows_b, sem_a, sem_b):
    wid = lax.axis_index("s") * _NC + lax.axis_index("c")
    s_base = pl.multiple_of(wid * _SELF_PW, _SELF_PW)
    n_base = pl.multiple_of(wid * _NB_PW, _NB_PW)

    pltpu.sync_copy(iself.at[pl.ds(s_base, _SELF_PW)], idx_s)
    da = [pltpu.async_copy(hb.at[idx_s.at[pl.ds(j * _LANES, _LANES)]],
                           rows_a.at[pl.ds(j * _LANES, _LANES)], sem_a)
          for j in range(_SELF_IR)]
    db = [pltpu.async_copy(hu.at[idx_s.at[pl.ds(j * _LANES, _LANES)]],
                           rows_b.at[pl.ds(j * _LANES, _LANES)], sem_b)
          for j in range(_SELF_IR)]
    for d in da:
        d.wait()
    pltpu.sync_copy(rows_a.at[pl.ds(0, _SELF_PW)],
                    sb.at[pl.ds(s_base, _SELF_PW)])
    for d in db:
        d.wait()
    pltpu.sync_copy(rows_b.at[pl.ds(0, _SELF_PW)],
                    su.at[pl.ds(s_base, _SELF_PW)])

    for itab, tbl, out in ((ipos, hb, nbp), (ineg, hu, nbn)):
        pltpu.sync_copy(itab.at[pl.ds(n_base, _NB_PW)], idx_n)

        def pair(p, carry):
            off_a = pl.multiple_of(p * (2 * _CHUNK), 2 * _CHUNK)
            off_b = off_a + _CHUNK
            da = _fire(tbl, idx_n, off_a, rows_a, sem_a)
            db = _fire(tbl, idx_n, off_b, rows_b, sem_b)
            for d in da:
                d.wait()
            pltpu.sync_copy(rows_a, out.at[pl.ds(n_base + off_a, _CHUNK)])
            for d in db:
                d.wait()
            pltpu.sync_copy(rows_b, out.at[pl.ds(n_base + off_b, _CHUNK)])
            return carry

        lax.fori_loop(0, _NB_PW // (2 * _CHUNK), pair, 0)


def _gather(hb, hu, iself, ipos, ineg):
    mesh = plsc.VectorSubcoreMesh(core_axis_name="c", subcore_axis_name="s")
    f = pl.kernel(
        _gather_body,
        mesh=mesh,
        out_type=[
            jax.ShapeDtypeStruct((_SELF_PAD, _DOUT), jnp.float32),
            jax.ShapeDtypeStruct((_SELF_PAD, _DOUT), jnp.float32),
            jax.ShapeDtypeStruct((_NB_PAD, _DOUT), jnp.float32),
            jax.ShapeDtypeStruct((_NB_PAD, _DOUT), jnp.float32),
        ],
        scratch_types=[
            pltpu.VMEM((_SELF_PW,), jnp.int32),
            pltpu.VMEM((_NB_PW,), jnp.int32),
            pltpu.VMEM((_CHUNK, _DOUT), jnp.float32),
            pltpu.VMEM((_CHUNK, _DOUT), jnp.float32),
            pltpu.SemaphoreType.DMA,
            pltpu.SemaphoreType.DMA,
        ],
        compiler_params=pltpu.CompilerParams(use_tc_tiling_on_sc=False),
    )
    return f(hb, hu, iself, ipos, ineg)


_SEG = _LANES // _DOUT
_BPAD = 10240
_TB = _BPAD // _SEG


def _attn_body(hsb, hsu, npos, nneg, a1b, a2b, a1u, a2u,
               xb, xu, rtb, rtu, dnb, dnu):
    j = pl.program_id(0)
    seg = (lax.broadcasted_iota(jnp.int32, (_SEG, _LANES), 1) // _DOUT
           == lax.broadcasted_iota(jnp.int32, (_SEG, _LANES), 0))
    expand = seg.astype(jnp.float32)

    @pl.when(j == 0)
    def _init():
        for hs, a1, a2, rt, dn, x in ((hsb, a1b, a2b, rtb, dnb, xb),
                                      (hsu, a1u, a2u, rtu, dnu, xu)):
            h = hs[...]
            r = jnp.dot(h, a1[...], preferred_element_type=jnp.float32)
            e = jnp.exp(-_leaky(
                r + jnp.dot(h, a2[...], preferred_element_type=jnp.float32)))
            rt[...] = r
            dn[...] = e
            x[...] = jnp.dot(e, expand,
                             preferred_element_type=jnp.float32) * h

    for nb, a2, rt, dn, x in ((npos, a2b, rtb, dnb, xb),
                              (nneg, a2u, rtu, dnu, xu)):
        h = nb[...]
        e = jnp.exp(-_leaky(
            rt[...] + jnp.dot(h, a2[...],
                              preferred_element_type=jnp.float32)))
        dn[...] += e
        x[...] += jnp.dot(e, expand, preferred_element_type=jnp.float32) * h

    @pl.when(j == _DEG - 1)
    def _final():
        for x, dn in ((xb, dnb), (xu, dnu)):
            d = jnp.dot(dn[...] + 1e-16, expand,
                        preferred_element_type=jnp.float32)
            v = x[...] / d
            x[...] = jnp.where(v > 0, v, jnp.exp(v) - 1.0)


def _attention(hsb, hsu, npos, nneg, a1b, a2b, a1u, a2u):
    return pl.pallas_call(
        _attn_body,
        grid=(_DEG,),
        in_specs=[
            pl.BlockSpec((_TB, _LANES), lambda j: (0, 0)),
            pl.BlockSpec((_TB, _LANES), lambda j: (0, 0)),
            pl.BlockSpec((_TB, _LANES), lambda j: (j, 0)),
            pl.BlockSpec((_TB, _LANES), lambda j: (j, 0)),
            pl.BlockSpec((_DIN, _SEG), lambda j: (0, 0)),
            pl.BlockSpec((_DIN, _SEG), lambda j: (0, 0)),
            pl.BlockSpec((_DIN, _SEG), lambda j: (0, 0)),
            pl.BlockSpec((_DIN, _SEG), lambda j: (0, 0)),
        ],
        out_specs=[
            pl.BlockSpec((_TB, _LANES), lambda j: (0, 0)),
            pl.BlockSpec((_TB, _LANES), lambda j: (0, 0)),
        ],
        out_shape=[
            jax.ShapeDtypeStruct((_TB, _LANES), jnp.float32),
            jax.ShapeDtypeStruct((_TB, _LANES), jnp.float32),
        ],
        scratch_shapes=[pltpu.VMEM((_TB, _SEG), jnp.float32)] * 4,
    )(hsb, hsu, npos, nneg, a1b, a2b, a1u, a2u)


def kernel(nodes, neigh_pos, neigh_neg, feat_table,
           W_bal, a_bal, W_unbal, a_unbal):
    hb, hu = _project(feat_table, W_bal, W_unbal)

    zs = jnp.zeros(_SELF_PAD - _B, jnp.int32)
    zc = jnp.zeros((_DEG, _BPAD - _B), jnp.int32)
    iself = jnp.concatenate([nodes, zs])
    ipos = jnp.concatenate([neigh_pos.T, zc], axis=1).reshape(-1)
    ineg = jnp.concatenate([neigh_neg.T, zc], axis=1).reshape(-1)

    sb, su, nbp, nbn = _gather(hb, hu, iself, ipos, ineg)

    hs2b = sb.reshape(-1, _LANES)[:_TB]
    hs2u = su.reshape(-1, _LANES)[:_TB]
    np2 = nbp.reshape(-1, _LANES)
    nn2 = nbn.reshape(-1, _LANES)

    eye = jnp.eye(_SEG, dtype=jnp.float32)
    a1b = jnp.kron(eye, a_bal[0, :_DOUT][:, None])
    a2b = jnp.kron(eye, a_bal[0, _DOUT:][:, None])
    a1u = jnp.kron(eye, a_unbal[0, :_DOUT][:, None])
    a2u = jnp.kron(eye, a_unbal[0, _DOUT:][:, None])

    xb2, xu2 = _attention(hs2b, hs2u, np2, nn2, a1b, a2b, a1u, a2u)
    return (xb2.reshape(-1, _DOUT)[:_B], xu2.reshape(-1, _DOUT)[:_B])
